# baseline (speedup 1.0000x reference)
import functools

import jax
import jax.numpy as jnp
from jax import lax
from jax.experimental import pallas as pl
from jax.experimental.pallas import tpu as pltpu
from jax.experimental.pallas import tpu_sc as plsc

BATCH = 4
SEQ = 4096
DIM = 2048
NTOK = BATCH * SEQ
NC = 2
NS = 16
NW = NC * NS
PER_W = NTOK // NW
WPR = SEQ // PER_W
LANES = 16
CH = 8
NCH = PER_W // CH
NB = 4
NVREG = PER_W // LANES


def _body(x_hbm, embed_hbm, out_hbm, x_v, idx_v, rows_bufs, gsems, wsems):
    wid = lax.axis_index("s") * NC + lax.axis_index("c")
    row = wid // WPR
    ch = wid % WPR

    pltpu.sync_copy(x_hbm.at[pl.ds(row * SEQ, SEQ)], x_v)

    def pre_body(i, carry):
        v = x_v[pl.ds(i * LANES, LANES)]
        ones = jnp.where(v != 0, 1, 0).astype(jnp.int32)
        return carry + jnp.sum(ones)

    carry0 = lax.fori_loop(0, ch * NVREG, pre_body, jnp.int32(0))

    base = ch * PER_W

    def pos_body(j, carry):
        v = x_v[pl.ds(base + j * LANES, LANES)]
        ones = jnp.where(v != 0, 1, 0).astype(jnp.int32)
        cs = jnp.cumsum(ones) + carry
        idx_v[pl.ds(j * LANES, LANES)] = cs * ones
        return carry + jnp.sum(ones)

    lax.fori_loop(0, NVREG, pos_body, carry0)

    out_base = wid * PER_W

    def start_gather(g, b):
        pltpu.async_copy(embed_hbm.at[idx_v.at[pl.ds(g * CH, CH)]],
                         rows_bufs[b], gsems[b])

    def wait_gather(b):
        pltpu.make_async_copy(embed_hbm.at[idx_v.at[pl.ds(0, CH)]],
                              rows_bufs[b], gsems[b]).wait()

    def start_write(g, b):
        pltpu.async_copy(rows_bufs[b],
                         out_hbm.at[pl.ds(out_base + g * CH, CH)], wsems[b])

    def wait_write(b):
        pltpu.make_async_copy(rows_bufs[b],
                              out_hbm.at[pl.ds(out_base, CH)],
                              wsems[b]).wait()

    start_gather(0, 0)
    start_gather(1, 1)

    def step(g, b, first, last):
        wait_gather(b)
        start_write(g, b)
        if not last:
            if not first:
                wait_write((b + 2) % NB)
            start_gather(g + 2, (b + 2) % NB)

    step(0, 0, True, False)
    step(1, 1, True, False)

    def g_body(k, _):
        g0 = 2 + 4 * k
        for j in range(4):
            step(g0 + j, (2 + j) % NB, False, False)
        return 0

    lax.fori_loop(0, (NCH - 4) // 4, g_body, 0)
    step(NCH - 2, (NCH - 2) % NB, False, True)
    step(NCH - 1, (NCH - 1) % NB, False, True)
    for b in range(NB):
        wait_write(b)


@jax.jit
def kernel(x, embed):
    x_flat = x.reshape(NTOK)
    mesh = plsc.VectorSubcoreMesh(
        core_axis_name="c", subcore_axis_name="s", num_cores=NC,
        num_subcores=NS,
    )
    out = pl.kernel(
        _body,
        out_type=jax.ShapeDtypeStruct((NTOK, DIM), jnp.float32),
        mesh=mesh,
        compiler_params=pltpu.CompilerParams(needs_layout_passes=False),
        scratch_types=[
            pltpu.VMEM((SEQ,), jnp.int32),
            pltpu.VMEM((PER_W,), jnp.int32),
            tuple(pltpu.VMEM((CH, DIM), jnp.float32) for _ in range(NB)),
            tuple(pltpu.SemaphoreType.DMA for _ in range(NB)),
            tuple(pltpu.SemaphoreType.DMA for _ in range(NB)),
        ],
    )(x_flat, embed)
    return out.reshape(BATCH, SEQ, DIM)
